# baseline (speedup 1.0000x reference)
import functools

import jax
import jax.numpy as jnp
import numpy as np
from jax import lax
from jax.experimental import pallas as pl
from jax.experimental.pallas import tpu as pltpu
from jax.experimental.pallas import tpu_sc as plsc

_B, _N, _C, _D = 4, 4096, 20, 64
_DP = 128
_NPOS = 1024
_T = 0.07
_HALF = _N // 2
_MAXIT = 34
_ITPAD = 48
_NC, _NS, _L = 2, 16, 16
_NW = _NC * _NS
_ROWS_PER_W = _B * _NPOS // (_NW // 2)
_NPAIR = _B * _C
_NQ = _C * _ITPAD
_OFF_K = _B * _NPOS
_OFF_QS = 9 * _NQ
_OFF_KS = 13 * _NQ
_ROWS_ALL = 17 * _NQ
_QROWS_W = _B * _NPOS // 8
_SROWS_W = _NPAIR * _ITPAD // 8

_SC_PARAMS = pltpu.CompilerParams(use_tc_tiling_on_sc=False,
                                  needs_layout_passes=False)


def _sc_mesh():
    return plsc.VectorSubcoreMesh(core_axis_name="c", subcore_axis_name="s",
                                  num_cores=_NC, num_subcores=_NS)


@functools.lru_cache(maxsize=1)
def _rng_consts_np():
    with jax.ensure_compile_time_eval():
        qb_idx, u1t, u2t = _rng_consts()
        return np.asarray(qb_idx), np.asarray(u1t), np.asarray(u2t)


def _rng_consts():
    idx_list = []
    for b in range(_B):
        skey = jax.random.fold_in(jax.random.key(42), b)
        inds = jax.random.choice(skey, _HALF, shape=(_NPOS,), replace=False)
        idx_list.append(inds.astype(jnp.int32) + b * _HALF)
    qb_idx = jnp.concatenate(idx_list)

    ns = (jnp.arange(_B, dtype=jnp.int32)[:, None] * 1000003
          + 2 * jnp.arange(_MAXIT, dtype=jnp.int32)[None, :]).ravel()

    def _u(n):
        return jax.random.uniform(jax.random.fold_in(jax.random.key(7), n), (_C,))

    u1 = jax.vmap(_u)(ns).reshape(_B, _MAXIT, _C)
    u2 = jax.vmap(_u)(ns + 1).reshape(_B, _MAXIT, _C)
    pad = ((0, 0), (0, 0), (0, _ITPAD - _MAXIT))
    u1t = jnp.pad(jnp.transpose(u1, (0, 2, 1)), pad)
    u2t = jnp.pad(jnp.transpose(u2, (0, 2, 1)), pad)
    return qb_idx, u1t, u2t


def _sc1_body(t1h_hbm, u1t_hbm, u2t_hbm,
              cnt_out, qsidx_out, ksidx_out,
              t_v, pos_v, u_v, qidx_v, cs_v):
    wid = lax.axis_index("s") * _NC + lax.axis_index("c")

    def _pair(p):
        b = p // _C
        c = p - b * _C
        pltpu.sync_copy(t1h_hbm.at[b], t_v)
        c_splat = jnp.full((_L,), 0, jnp.int32) + c
        base = b * _HALF

        def scan_chunk(idx, woff):
            tch = t_v[pl.ds(idx * _L, _L)]
            lane = lax.iota(jnp.int32, _L)
            mi = 1 - jnp.minimum(jnp.abs(tch - c_splat), 1)
            incl = plsc.cumsum(mi)
            tgt = mi * (woff + incl - 1) + (1 - mi) * (_HALF + lane)
            plsc.store_scatter(pos_v, [tgt], lane + (idx * _L + base))
            dn = lax.GatherDimensionNumbers(offset_dims=(),
                                            collapsed_slice_dims=(0,),
                                            start_index_map=(0,))
            tot = lax.gather(incl, jnp.full((_L, 1), _L - 1, jnp.int32),
                             dimension_numbers=dn, slice_sizes=(1,),
                             mode=lax.GatherScatterMode.PROMISE_IN_BOUNDS)
            return woff + tot

        def chunk_body(i, woff):
            for u in range(8):
                woff = scan_chunk(i * 8 + u, woff)
            return woff

        cnt = lax.fori_loop(0, _HALF // _L // 8, chunk_body,
                            jnp.full((_L,), 0, jnp.int32))
        cs_v[...] = cnt
        pltpu.sync_copy(cs_v, cnt_out.at[b, c])
        cntf = cnt.astype(jnp.float32)

        def _resolve(ut_hbm, idx_out):
            pltpu.sync_copy(ut_hbm.at[b, c], u_v)
            for k3 in range(_ITPAD // _L):
                u = u_v[pl.ds(k3 * _L, _L)]
                off = jnp.clip((u * cntf).astype(jnp.int32), 0, _HALF - 1)
                jrow = plsc.load_gather(pos_v, [off])
                qidx_v[pl.ds(k3 * _L, _L)] = jnp.clip(jrow, 0, _B * _HALF - 1)
            pltpu.sync_copy(qidx_v, idx_out.at[pl.ds(p * _ITPAD, _ITPAD)])

        _resolve(u1t_hbm, qsidx_out)
        _resolve(u2t_hbm, ksidx_out)

    for k in range(3):
        p = wid + _NW * k

        @pl.when(p < _NPAIR)
        def _():
            _pair(p)


def _sc2_body(feat1_hbm, feat2_hbm, qbidx_hbm, qsidx_hbm, ksidx_hbm,
              rows_out, idxq_v, idxs_v, rows_v, sem):
    wid = lax.axis_index("s") * _NC + lax.axis_index("c")
    role = wid // 8
    j = lax.rem(wid, 8)

    def _gather(idx_hbm, n, feat_hbm, dst0):
        r0 = j * n
        idx_v = idxq_v if n == _QROWS_W else idxs_v
        dst_v = rows_v.at[pl.ds(0, n)] if n != _QROWS_W else rows_v
        pltpu.sync_copy(idx_hbm.at[pl.ds(r0, n)], idx_v)
        pltpu.async_copy(feat_hbm.at[idx_v], dst_v, sem).wait()
        pltpu.sync_copy(dst_v, rows_out.at[pl.ds(dst0 + r0, n)])

    @pl.when(role == 0)
    def _():
        _gather(qbidx_hbm, _QROWS_W, feat1_hbm, 0)

    @pl.when(role == 1)
    def _():
        _gather(qbidx_hbm, _QROWS_W, feat2_hbm, _OFF_K)

    @pl.when(role == 2)
    def _():
        _gather(qsidx_hbm, _SROWS_W, feat1_hbm, _OFF_QS)

    @pl.when(role == 3)
    def _():
        _gather(ksidx_hbm, _SROWS_W, feat2_hbm, _OFF_KS)


def _sc1_call(t1h, u1t, u2t):
    out_type = [
        jax.ShapeDtypeStruct((_B, _C, _L), jnp.int32),
        jax.ShapeDtypeStruct((_NPAIR * _ITPAD,), jnp.int32),
        jax.ShapeDtypeStruct((_NPAIR * _ITPAD,), jnp.int32),
    ]
    scratch = [
        pltpu.VMEM((_HALF,), jnp.int32),
        pltpu.VMEM((_HALF + _L,), jnp.int32),
        pltpu.VMEM((_ITPAD,), jnp.float32),
        pltpu.VMEM((_ITPAD,), jnp.int32),
        pltpu.VMEM((_L,), jnp.int32),
    ]
    fn = pl.kernel(_sc1_body, out_type=out_type, mesh=_sc_mesh(),
                   scratch_types=scratch, compiler_params=_SC_PARAMS)
    return fn(t1h, u1t, u2t)


def _sc2_call(feat1_flat, feat2_flat, qb_idx, qsidx, ksidx):
    out_type = jax.ShapeDtypeStruct((_ROWS_ALL, _DP), jnp.float32)
    scratch = [
        pltpu.VMEM((_QROWS_W,), jnp.int32),
        pltpu.VMEM((_SROWS_W,), jnp.int32),
        pltpu.VMEM((_QROWS_W, _DP), jnp.float32),
        pltpu.SemaphoreType.DMA,
    ]
    fn = pl.kernel(_sc2_body, out_type=out_type, mesh=_sc_mesh(),
                   scratch_types=scratch,
                   compiler_params=pltpu.CompilerParams(
                       use_tc_tiling_on_sc=True,
                       needs_layout_passes=False))
    return fn(feat1_flat, feat2_flat, qb_idx, qsidx, ksidx)


def _tc_seg_body(seg_ref, tgt_ref, out_ref):
    total = jnp.float32(0.0)
    for b in range(_B):
        seg = seg_ref[b]
        m = jnp.max(seg, axis=0, keepdims=True)
        lse = jnp.log(jnp.sum(jnp.exp(seg - m), axis=0, keepdims=True)) + m
        tgt = tgt_ref[pl.ds(b, 1), :]
        onehot = lax.broadcasted_iota(jnp.int32, (_C, _N), 0) == tgt
        picked = jnp.sum(jnp.where(onehot, seg, 0.0), axis=0, keepdims=True)
        total += 0.02 * jnp.sum(lse - picked) / (_B * _N)
    out_ref[0, 0] = total


def _tc_main_body(q_ref, k_ref, qs_ref, ks_ref, cnt_ref, seg_part_ref,
                  out_ref):
    b = pl.program_id(0)

    @pl.when(b == 0)
    def _():
        out_ref[0, 0] = seg_part_ref[0, 0]

    def _norm(x):
        return x / jnp.sqrt(jnp.sum(x * x, axis=1, keepdims=True))

    def _nll(q, k, mask=None):
        qn, kn = _norm(q), _norm(k)
        z = lax.dot_general(qn, kn, (((1,), (1,)), ((), ())),
                            preferred_element_type=jnp.float32) / _T
        zm = z if mask is None else jnp.where(mask, z, -1e30)
        m = jnp.max(zm, axis=1, keepdims=True)
        lse = jnp.log(jnp.sum(jnp.exp(zm - m), axis=1, keepdims=True)) + m
        diag = jnp.sum(qn * kn, axis=1, keepdims=True) / _T
        return lse - diag

    total = 0.1 * jnp.sum(_nll(q_ref[...], k_ref[...])) / (_NPOS * _B)

    rit = lax.rem(lax.broadcasted_iota(jnp.int32, (_NQ, 1), 0), _ITPAD)
    cit = lax.rem(lax.broadcasted_iota(jnp.int32, (1, _NQ), 1), _ITPAD)
    min_it = jnp.min(cnt_ref[b]) // 3
    nll = _nll(qs_ref[...], ks_ref[...], mask=rit == cit)
    li = jnp.sum(jnp.where(rit < min_it, nll, 0.0))
    total += li / (_C * min_it.astype(jnp.float32)) / _B

    out_ref[0, 0] += total


def kernel(_feat1, _feat2, seg_pred1, seg_pred2, target1, target2):
    try:
        qb_idx, u1t, u2t = (jnp.asarray(x) for x in _rng_consts_np())
    except Exception:
        qb_idx, u1t, u2t = _rng_consts()
    lanepad = ((0, 0), (0, 0), (0, _DP - _D))
    feat1_flat = jnp.pad(_feat1[:, :_HALF, :], lanepad).reshape(_B * _HALF, _DP)
    feat2_flat = jnp.pad(_feat2[:, :_HALF, :], lanepad).reshape(_B * _HALF, _DP)
    t1h = target1[:, :_HALF]

    cnts, qsidx, ksidx = _sc1_call(t1h, u1t, u2t)
    rows = _sc2_call(feat1_flat, feat2_flat, qb_idx, qsidx, ksidx)

    seg_part = pl.pallas_call(
        _tc_seg_body,
        out_shape=jax.ShapeDtypeStruct((1, 1), jnp.float32),
        out_specs=pl.BlockSpec(memory_space=pltpu.SMEM),
    )(seg_pred1, target1)

    out = pl.pallas_call(
        _tc_main_body,
        grid=(_B,),
        out_shape=jax.ShapeDtypeStruct((1, 1), jnp.float32),
        in_specs=[
            pl.BlockSpec((_NPOS, _DP), lambda b: (b, 0)),
            pl.BlockSpec((_NPOS, _DP), lambda b: (_OFF_K // _NPOS + b, 0)),
            pl.BlockSpec((_NQ, _DP), lambda b: (_OFF_QS // _NQ + b, 0)),
            pl.BlockSpec((_NQ, _DP), lambda b: (_OFF_KS // _NQ + b, 0)),
            pl.BlockSpec(memory_space=pltpu.VMEM),
            pl.BlockSpec(memory_space=pltpu.SMEM),
        ],
        out_specs=pl.BlockSpec(memory_space=pltpu.SMEM),
    )(rows, rows, rows, rows, cnts, seg_part)
    return out[0, 0]


# speedup vs baseline: 1.0136x; 1.0136x over previous
import functools

import jax
import jax.numpy as jnp
import numpy as np
from jax import lax
from jax.experimental import pallas as pl
from jax.experimental.pallas import tpu as pltpu
from jax.experimental.pallas import tpu_sc as plsc

_B, _N, _C, _D = 4, 4096, 20, 64
_DP = 128
_NPOS = 1024
_T = 0.07
_HALF = _N // 2
_MAXIT = 34
_ITPAD = 48
_NC, _NS, _L = 2, 16, 16
_NW = _NC * _NS
_ROWS_PER_W = _B * _NPOS // (_NW // 2)
_NPAIR = _B * _C
_NQ = _C * _ITPAD
_OFF_K = _B * _NPOS
_OFF_QS = 9 * _NQ
_OFF_KS = 13 * _NQ
_ROWS_ALL = 17 * _NQ
_QROWS_W = _B * _NPOS // 8
_SROWS_W = _NPAIR * _ITPAD // 8

_SC_PARAMS = pltpu.CompilerParams(use_tc_tiling_on_sc=False,
                                  needs_layout_passes=False)


def _sc_mesh():
    return plsc.VectorSubcoreMesh(core_axis_name="c", subcore_axis_name="s",
                                  num_cores=_NC, num_subcores=_NS)


@functools.lru_cache(maxsize=1)
def _rng_consts_np():
    with jax.ensure_compile_time_eval():
        qb_idx, u1t, u2t = _rng_consts()
        return np.asarray(qb_idx), np.asarray(u1t), np.asarray(u2t)


def _rng_consts():
    idx_list = []
    for b in range(_B):
        skey = jax.random.fold_in(jax.random.key(42), b)
        inds = jax.random.choice(skey, _HALF, shape=(_NPOS,), replace=False)
        idx_list.append(inds.astype(jnp.int32) + b * _HALF)
    qb_idx = jnp.concatenate(idx_list)

    ns = (jnp.arange(_B, dtype=jnp.int32)[:, None] * 1000003
          + 2 * jnp.arange(_MAXIT, dtype=jnp.int32)[None, :]).ravel()

    def _u(n):
        return jax.random.uniform(jax.random.fold_in(jax.random.key(7), n), (_C,))

    u1 = jax.vmap(_u)(ns).reshape(_B, _MAXIT, _C)
    u2 = jax.vmap(_u)(ns + 1).reshape(_B, _MAXIT, _C)
    pad = ((0, 0), (0, 0), (0, _ITPAD - _MAXIT))
    u1t = jnp.pad(jnp.transpose(u1, (0, 2, 1)), pad)
    u2t = jnp.pad(jnp.transpose(u2, (0, 2, 1)), pad)
    return qb_idx, u1t, u2t


def _sc1_body(t1h_hbm, u1t_hbm, u2t_hbm,
              cnt_out, qsidx_out, ksidx_out,
              t_v, pos_v, u_v, qidx_v, cs_v):
    wid = lax.axis_index("s") * _NC + lax.axis_index("c")

    def _pair(p):
        b = p // _C
        c = p - b * _C
        pltpu.sync_copy(t1h_hbm.at[b], t_v)
        c_splat = jnp.full((_L,), 0, jnp.int32) + c
        base = b * _HALF

        def scan_chunk(idx, woff):
            tch = t_v[pl.ds(idx * _L, _L)]
            lane = lax.iota(jnp.int32, _L)
            mi = 1 - jnp.minimum(jnp.abs(tch - c_splat), 1)
            incl = plsc.cumsum(mi)
            tgt = mi * (woff + incl - 1) + (1 - mi) * (_HALF + lane)
            plsc.store_scatter(pos_v, [tgt], lane + (idx * _L + base))
            dn = lax.GatherDimensionNumbers(offset_dims=(),
                                            collapsed_slice_dims=(0,),
                                            start_index_map=(0,))
            tot = lax.gather(incl, jnp.full((_L, 1), _L - 1, jnp.int32),
                             dimension_numbers=dn, slice_sizes=(1,),
                             mode=lax.GatherScatterMode.PROMISE_IN_BOUNDS)
            return woff + tot

        def chunk_body(i, woff):
            for u in range(4):
                woff = scan_chunk(i * 4 + u, woff)
            return woff

        cnt = lax.fori_loop(0, _HALF // _L // 4, chunk_body,
                            jnp.full((_L,), 0, jnp.int32))
        cs_v[...] = cnt
        pltpu.sync_copy(cs_v, cnt_out.at[b, c])
        cntf = cnt.astype(jnp.float32)

        def _resolve(ut_hbm, idx_out):
            pltpu.sync_copy(ut_hbm.at[b, c], u_v)
            for k3 in range(_ITPAD // _L):
                u = u_v[pl.ds(k3 * _L, _L)]
                off = jnp.clip((u * cntf).astype(jnp.int32), 0, _HALF - 1)
                jrow = plsc.load_gather(pos_v, [off])
                qidx_v[pl.ds(k3 * _L, _L)] = jnp.clip(jrow, 0, _B * _HALF - 1)
            pltpu.sync_copy(qidx_v, idx_out.at[pl.ds(p * _ITPAD, _ITPAD)])

        _resolve(u1t_hbm, qsidx_out)
        _resolve(u2t_hbm, ksidx_out)

    for k in range(3):
        p = wid + _NW * k

        @pl.when(p < _NPAIR)
        def _():
            _pair(p)


def _sc2_body(feat1_hbm, feat2_hbm, qbidx_hbm, qsidx_hbm, ksidx_hbm,
              rows_out, idxq_v, idxs_v, rows_v, sem):
    wid = lax.axis_index("s") * _NC + lax.axis_index("c")
    role = wid // 8
    j = lax.rem(wid, 8)

    def _gather(idx_hbm, n, feat_hbm, dst0):
        r0 = j * n
        idx_v = idxq_v if n == _QROWS_W else idxs_v
        dst_v = rows_v.at[pl.ds(0, n)] if n != _QROWS_W else rows_v
        pltpu.sync_copy(idx_hbm.at[pl.ds(r0, n)], idx_v)
        pltpu.async_copy(feat_hbm.at[idx_v], dst_v, sem).wait()
        pltpu.sync_copy(dst_v, rows_out.at[pl.ds(dst0 + r0, n)])

    @pl.when(role == 0)
    def _():
        _gather(qbidx_hbm, _QROWS_W, feat1_hbm, 0)

    @pl.when(role == 1)
    def _():
        _gather(qbidx_hbm, _QROWS_W, feat2_hbm, _OFF_K)

    @pl.when(role == 2)
    def _():
        _gather(qsidx_hbm, _SROWS_W, feat1_hbm, _OFF_QS)

    @pl.when(role == 3)
    def _():
        _gather(ksidx_hbm, _SROWS_W, feat2_hbm, _OFF_KS)


def _sc1_call(t1h, u1t, u2t):
    out_type = [
        jax.ShapeDtypeStruct((_B, _C, _L), jnp.int32),
        jax.ShapeDtypeStruct((_NPAIR * _ITPAD,), jnp.int32),
        jax.ShapeDtypeStruct((_NPAIR * _ITPAD,), jnp.int32),
    ]
    scratch = [
        pltpu.VMEM((_HALF,), jnp.int32),
        pltpu.VMEM((_HALF + _L,), jnp.int32),
        pltpu.VMEM((_ITPAD,), jnp.float32),
        pltpu.VMEM((_ITPAD,), jnp.int32),
        pltpu.VMEM((_L,), jnp.int32),
    ]
    fn = pl.kernel(_sc1_body, out_type=out_type, mesh=_sc_mesh(),
                   scratch_types=scratch, compiler_params=_SC_PARAMS)
    return fn(t1h, u1t, u2t)


def _sc2_call(feat1_flat, feat2_flat, qb_idx, qsidx, ksidx):
    out_type = jax.ShapeDtypeStruct((_ROWS_ALL, _DP), jnp.float32)
    scratch = [
        pltpu.VMEM((_QROWS_W,), jnp.int32),
        pltpu.VMEM((_SROWS_W,), jnp.int32),
        pltpu.VMEM((_QROWS_W, _DP), jnp.float32),
        pltpu.SemaphoreType.DMA,
    ]
    fn = pl.kernel(_sc2_body, out_type=out_type, mesh=_sc_mesh(),
                   scratch_types=scratch,
                   compiler_params=pltpu.CompilerParams(
                       use_tc_tiling_on_sc=True,
                       needs_layout_passes=False))
    return fn(feat1_flat, feat2_flat, qb_idx, qsidx, ksidx)


def _tc_seg_body(seg_ref, tgt_ref, out_ref):
    total = jnp.float32(0.0)
    for b in range(_B):
        seg = seg_ref[b]
        m = jnp.max(seg, axis=0, keepdims=True)
        lse = jnp.log(jnp.sum(jnp.exp(seg - m), axis=0, keepdims=True)) + m
        tgt = tgt_ref[pl.ds(b, 1), :]
        onehot = lax.broadcasted_iota(jnp.int32, (_C, _N), 0) == tgt
        picked = jnp.sum(jnp.where(onehot, seg, 0.0), axis=0, keepdims=True)
        total += 0.02 * jnp.sum(lse - picked) / (_B * _N)
    out_ref[0, 0] = total


def _tc_main_body(q_ref, k_ref, qs_ref, ks_ref, cnt_ref, seg_part_ref,
                  out_ref):
    b = pl.program_id(0)

    @pl.when(b == 0)
    def _():
        out_ref[0, 0] = seg_part_ref[0, 0]

    def _norm(x):
        return x / jnp.sqrt(jnp.sum(x * x, axis=1, keepdims=True))

    def _nll(q, k, mask=None):
        qn, kn = _norm(q), _norm(k)
        z = lax.dot_general(qn, kn, (((1,), (1,)), ((), ())),
                            preferred_element_type=jnp.float32) / _T
        zm = z if mask is None else jnp.where(mask, z, -1e30)
        m = jnp.max(zm, axis=1, keepdims=True)
        lse = jnp.log(jnp.sum(jnp.exp(zm - m), axis=1, keepdims=True)) + m
        diag = jnp.sum(qn * kn, axis=1, keepdims=True) / _T
        return lse - diag

    total = 0.1 * jnp.sum(_nll(q_ref[...], k_ref[...])) / (_NPOS * _B)

    rit = lax.rem(lax.broadcasted_iota(jnp.int32, (_NQ, 1), 0), _ITPAD)
    cit = lax.rem(lax.broadcasted_iota(jnp.int32, (1, _NQ), 1), _ITPAD)
    min_it = jnp.min(cnt_ref[b]) // 3
    nll = _nll(qs_ref[...], ks_ref[...], mask=rit == cit)
    li = jnp.sum(jnp.where(rit < min_it, nll, 0.0))
    total += li / (_C * min_it.astype(jnp.float32)) / _B

    out_ref[0, 0] += total


def kernel(_feat1, _feat2, seg_pred1, seg_pred2, target1, target2):
    try:
        qb_idx, u1t, u2t = (jnp.asarray(x) for x in _rng_consts_np())
    except Exception:
        qb_idx, u1t, u2t = _rng_consts()
    lanepad = ((0, 0), (0, 0), (0, _DP - _D))
    feat1_flat = jnp.pad(_feat1[:, :_HALF, :], lanepad).reshape(_B * _HALF, _DP)
    feat2_flat = jnp.pad(_feat2[:, :_HALF, :], lanepad).reshape(_B * _HALF, _DP)
    t1h = target1[:, :_HALF]

    cnts, qsidx, ksidx = _sc1_call(t1h, u1t, u2t)
    rows = _sc2_call(feat1_flat, feat2_flat, qb_idx, qsidx, ksidx)

    seg_part = pl.pallas_call(
        _tc_seg_body,
        out_shape=jax.ShapeDtypeStruct((1, 1), jnp.float32),
        out_specs=pl.BlockSpec(memory_space=pltpu.SMEM),
    )(seg_pred1, target1)

    out = pl.pallas_call(
        _tc_main_body,
        grid=(_B,),
        out_shape=jax.ShapeDtypeStruct((1, 1), jnp.float32),
        in_specs=[
            pl.BlockSpec((_NPOS, _DP), lambda b: (b, 0)),
            pl.BlockSpec((_NPOS, _DP), lambda b: (_OFF_K // _NPOS + b, 0)),
            pl.BlockSpec((_NQ, _DP), lambda b: (_OFF_QS // _NQ + b, 0)),
            pl.BlockSpec((_NQ, _DP), lambda b: (_OFF_KS // _NQ + b, 0)),
            pl.BlockSpec(memory_space=pltpu.VMEM),
            pl.BlockSpec(memory_space=pltpu.SMEM),
        ],
        out_specs=pl.BlockSpec(memory_space=pltpu.SMEM),
    )(rows, rows, rows, rows, cnts, seg_part)
    return out[0, 0]
